# Initial kernel scaffold; baseline (speedup 1.0000x reference)
import jax
import jax.numpy as jnp
from jax.experimental import pallas as pl


def kernel(x, W_lin, b_lin, W_lin1, b_lin1, W_max, b_max, W_aug, b_aug, gn1_w, gn1_b, gn2_w, gn2_b):
    raise NotImplementedError("write your pallas kernel here")


# speedup vs baseline: 1.0785x; 1.0785x over previous
import functools

import jax
import jax.numpy as jnp
from jax import lax
from jax.experimental import pallas as pl
from jax.experimental.pallas import tpu as pltpu
from jax.experimental.pallas import tpu_sc as plsc

K = 20
TL = 1024


def _k1_body(x_ref, wlT_ref, bl_ref, wl1T_ref, bl1_ref,
             x2_ref, sum_ref, max_ref):
    t = pl.program_id(1)
    x1 = jnp.dot(x_ref[0].astype(jnp.bfloat16), wlT_ref[...],
                 preferred_element_type=jnp.float32) + bl_ref[...]
    part_sum = jnp.sum(x1, axis=0, keepdims=True)
    part_max = jnp.max(x1, axis=0, keepdims=True)

    @pl.when(t == 0)
    def _():
        sum_ref[0] = part_sum
        max_ref[0] = part_max

    @pl.when(t != 0)
    def _():
        sum_ref[0] = sum_ref[0] + part_sum
        max_ref[0] = jnp.maximum(max_ref[0], part_max)

    x2_ref[0] = jnp.dot(x1.astype(jnp.bfloat16), wl1T_ref[...],
                        preferred_element_type=jnp.float32) + bl1_ref[...]


def _k2_body(sum_ref, max_ref, wmT_ref, bm_ref, waT_ref, ba_ref,
             g1w_ref, g1b_ref, g2w_ref, g2b_ref, agg_ref, aggT_ref,
             v_ref, *, L, groups, eps):
    cs = 1.0 / (agg_ref.shape[0] // groups)
    hi = lax.Precision.HIGHEST

    def gn(y, w, b):
        gs = jnp.dot(y, agg_ref[...], precision=hi,
                     preferred_element_type=jnp.float32) * cs
        gss = jnp.dot(y * y, agg_ref[...], precision=hi,
                      preferred_element_type=jnp.float32) * cs
        var = gss - gs * gs
        rstd = lax.rsqrt(var + eps)
        mean_f = jnp.dot(gs, aggT_ref[...], precision=hi,
                         preferred_element_type=jnp.float32)
        rstd_f = jnp.dot(rstd, aggT_ref[...], precision=hi,
                         preferred_element_type=jnp.float32)
        return (y - mean_f) * rstd_f * w + b

    mean = (sum_ref[...][:, 0, :] * (1.0 / L)).astype(jnp.bfloat16)
    xm = jnp.dot(mean, wmT_ref[...],
                 preferred_element_type=jnp.float32) + bm_ref[...]
    xa = jnp.dot(max_ref[...][:, 0, :].astype(jnp.bfloat16), waT_ref[...],
                 preferred_element_type=jnp.float32) + ba_ref[...]
    g1 = gn(xm, g1w_ref[...], g1b_ref[...]).astype(jnp.bfloat16)
    g2 = gn(xa, g2w_ref[...], g2b_ref[...]).astype(jnp.bfloat16)
    v_ref[:, 0, :] = (g1.astype(jnp.float32) + g2.astype(jnp.float32))


def _k3_body(x2_ref, v_ref, s_ref):
    x2b = x2_ref[0].astype(jnp.bfloat16).astype(jnp.float32)
    s_ref[0] = jnp.sum(x2b * v_ref[0], axis=1, keepdims=True)


def _k4_body(s_ref, idx_ref, *, L, k):
    s = s_ref[...]
    B = s.shape[0]
    col = lax.broadcasted_iota(jnp.int32, (B, L), 1)
    cols = []
    for _ in range(k):
        m = jnp.max(s, axis=1, keepdims=True)
        eq = s == m
        idx = jnp.min(jnp.where(eq, col, L), axis=1, keepdims=True)
        cols.append(idx)
        s = jnp.where(col == idx, -jnp.inf, s)
    idxmat = jnp.concatenate(cols, axis=1)
    idx_ref[...] = idxmat + lax.broadcasted_iota(jnp.int32, (B, k), 0) * L


def _sc_gather(table, idx_flat, n_rows, rows_per_worker):
    D = table.shape[1]
    n_workers = n_rows // rows_per_worker
    mesh = plsc.VectorSubcoreMesh(core_axis_name="c", subcore_axis_name="s")

    @functools.partial(
        pl.kernel, mesh=mesh,
        out_type=jax.ShapeDtypeStruct((n_rows, D), jnp.float32),
        scratch_types=[
            pltpu.VMEM((rows_per_worker,), jnp.int32),
            pltpu.VMEM((rows_per_worker, D), jnp.float32),
            pltpu.SemaphoreType.DMA,
        ],
    )
    def gather_k(table_hbm, idx_hbm, out_hbm, idx_v, rows_v, sem):
        wid = lax.axis_index("s") * 2 + lax.axis_index("c")

        @pl.when(wid < n_workers)
        def _():
            base = wid * rows_per_worker
            pltpu.sync_copy(idx_hbm.at[pl.ds(base, rows_per_worker)], idx_v)
            pltpu.async_copy(table_hbm.at[idx_v], rows_v, sem).wait()
            pltpu.sync_copy(rows_v, out_hbm.at[pl.ds(base, rows_per_worker)])

    return gather_k(table, idx_flat)


def kernel(x, W_lin, b_lin, W_lin1, b_lin1, W_max, b_max, W_aug, b_aug,
           gn1_w, gn1_b, gn2_w, gn2_b):
    B, L, D = x.shape
    NT = L // TL
    groups = 32

    wlT = W_lin.T.astype(jnp.bfloat16)
    wl1T = W_lin1.T.astype(jnp.bfloat16)
    wmT = W_max.T.astype(jnp.bfloat16)
    waT = W_aug.T.astype(jnp.bfloat16)
    row = lambda a: a.reshape(1, D)

    x2, colsum, colmax = pl.pallas_call(
        _k1_body,
        grid=(B, NT),
        in_specs=[
            pl.BlockSpec((1, TL, D), lambda b, t: (b, t, 0)),
            pl.BlockSpec((D, D), lambda b, t: (0, 0)),
            pl.BlockSpec((1, D), lambda b, t: (0, 0)),
            pl.BlockSpec((D, D), lambda b, t: (0, 0)),
            pl.BlockSpec((1, D), lambda b, t: (0, 0)),
        ],
        out_specs=[
            pl.BlockSpec((1, TL, D), lambda b, t: (b, t, 0)),
            pl.BlockSpec((1, 1, D), lambda b, t: (b, 0, 0)),
            pl.BlockSpec((1, 1, D), lambda b, t: (b, 0, 0)),
        ],
        out_shape=[
            jax.ShapeDtypeStruct((B, L, D), jnp.float32),
            jax.ShapeDtypeStruct((B, 1, D), jnp.float32),
            jax.ShapeDtypeStruct((B, 1, D), jnp.float32),
        ],
        compiler_params=pltpu.CompilerParams(
            dimension_semantics=("arbitrary", "arbitrary")),
    )(x, wlT, row(b_lin), wl1T, row(b_lin1))

    agg = (jnp.arange(D, dtype=jnp.int32)[:, None] // (D // groups)
           == jnp.arange(groups, dtype=jnp.int32)[None, :]).astype(jnp.float32)
    v = pl.pallas_call(
        functools.partial(_k2_body, L=L, groups=groups, eps=1e-5),
        out_shape=jax.ShapeDtypeStruct((B, 1, D), jnp.float32),
    )(colsum, colmax, wmT, row(b_max), waT, row(b_aug),
      row(gn1_w), row(gn1_b), row(gn2_w), row(gn2_b), agg, agg.T)

    s3 = pl.pallas_call(
        _k3_body,
        grid=(B, NT),
        in_specs=[
            pl.BlockSpec((1, TL, D), lambda b, t: (b, t, 0)),
            pl.BlockSpec((1, 1, D), lambda b, t: (b, 0, 0)),
        ],
        out_specs=pl.BlockSpec((1, TL, 1), lambda b, t: (b, t, 0)),
        out_shape=jax.ShapeDtypeStruct((B, L, 1), jnp.float32),
        compiler_params=pltpu.CompilerParams(
            dimension_semantics=("arbitrary", "arbitrary")),
    )(x2, v)
    s = s3.reshape(B, L)

    flat_idx = pl.pallas_call(
        functools.partial(_k4_body, L=L, k=K),
        out_shape=jax.ShapeDtypeStruct((B, K), jnp.int32),
    )(s)

    selected = _sc_gather(x2.reshape(B * L, D), flat_idx.reshape(B * K),
                          n_rows=B * K, rows_per_worker=8)
    return selected.reshape(B, K, D)


# speedup vs baseline: 1.1143x; 1.0332x over previous
import functools

import jax
import jax.numpy as jnp
from jax import lax
from jax.experimental import pallas as pl
from jax.experimental.pallas import tpu as pltpu
from jax.experimental.pallas import tpu_sc as plsc

K = 20
TL = 1024


def _k1_body(x_ref, wlT_ref, bl_ref, wl1T_ref, bl1_ref,
             x2_ref, sum_ref, max_ref):
    t = pl.program_id(1)
    x1 = jnp.dot(x_ref[0].astype(jnp.bfloat16), wlT_ref[...],
                 preferred_element_type=jnp.float32) + bl_ref[...]
    part_sum = jnp.sum(x1, axis=0, keepdims=True)
    part_max = jnp.max(x1, axis=0, keepdims=True)

    @pl.when(t == 0)
    def _():
        sum_ref[0] = part_sum
        max_ref[0] = part_max

    @pl.when(t != 0)
    def _():
        sum_ref[0] = sum_ref[0] + part_sum
        max_ref[0] = jnp.maximum(max_ref[0], part_max)

    x2 = jnp.dot(x1.astype(jnp.bfloat16), wl1T_ref[...],
                 preferred_element_type=jnp.float32) + bl1_ref[...]
    x2_ref[0] = x2.astype(jnp.bfloat16)


def _k2_body(sum_ref, max_ref, wmT_ref, bm_ref, waT_ref, ba_ref,
             g1w_ref, g1b_ref, g2w_ref, g2b_ref, agg_ref, aggT_ref,
             v_ref, *, L, groups, eps):
    cs = 1.0 / (agg_ref.shape[0] // groups)
    hi = lax.Precision.HIGHEST

    def gn(y, w, b):
        gs = jnp.dot(y, agg_ref[...], precision=hi,
                     preferred_element_type=jnp.float32) * cs
        gss = jnp.dot(y * y, agg_ref[...], precision=hi,
                      preferred_element_type=jnp.float32) * cs
        var = gss - gs * gs
        rstd = lax.rsqrt(var + eps)
        mean_f = jnp.dot(gs, aggT_ref[...], precision=hi,
                         preferred_element_type=jnp.float32)
        rstd_f = jnp.dot(rstd, aggT_ref[...], precision=hi,
                         preferred_element_type=jnp.float32)
        return (y - mean_f) * rstd_f * w + b

    mean = (sum_ref[...][:, 0, :] * (1.0 / L)).astype(jnp.bfloat16)
    xm = jnp.dot(mean, wmT_ref[...],
                 preferred_element_type=jnp.float32) + bm_ref[...]
    xa = jnp.dot(max_ref[...][:, 0, :].astype(jnp.bfloat16), waT_ref[...],
                 preferred_element_type=jnp.float32) + ba_ref[...]
    g1 = gn(xm, g1w_ref[...], g1b_ref[...]).astype(jnp.bfloat16)
    g2 = gn(xa, g2w_ref[...], g2b_ref[...]).astype(jnp.bfloat16)
    v_ref[:, 0, :] = (g1.astype(jnp.float32) + g2.astype(jnp.float32))


def _k3_body(x2_ref, v_ref, s_ref):
    x2b = x2_ref[0].astype(jnp.float32)
    s_ref[0] = jnp.sum(x2b * v_ref[0], axis=1, keepdims=True)


def _k6_body(xr_ref, wlT_ref, bl_ref, wl1T_ref, bl1_ref, out_ref):
    x1 = jnp.dot(xr_ref[...].astype(jnp.bfloat16), wlT_ref[...],
                 preferred_element_type=jnp.float32) + bl_ref[...]
    out_ref[...] = jnp.dot(x1.astype(jnp.bfloat16), wl1T_ref[...],
                           preferred_element_type=jnp.float32) + bl1_ref[...]


def _k4_body(s_ref, idx_ref, *, L, k):
    s = s_ref[...]
    B = s.shape[0]
    col = lax.broadcasted_iota(jnp.int32, (B, L), 1)
    cols = []
    for _ in range(k):
        m = jnp.max(s, axis=1, keepdims=True)
        eq = s == m
        idx = jnp.min(jnp.where(eq, col, L), axis=1, keepdims=True)
        cols.append(idx)
        s = jnp.where(col == idx, -jnp.inf, s)
    idxmat = jnp.concatenate(cols, axis=1)
    idx_ref[...] = idxmat + lax.broadcasted_iota(jnp.int32, (B, k), 0) * L


def _sc_gather(table, idx_flat, n_rows, rows_per_worker):
    D = table.shape[1]
    n_workers = n_rows // rows_per_worker
    mesh = plsc.VectorSubcoreMesh(core_axis_name="c", subcore_axis_name="s")

    @functools.partial(
        pl.kernel, mesh=mesh,
        out_type=jax.ShapeDtypeStruct((n_rows, D), jnp.float32),
        scratch_types=[
            pltpu.VMEM((rows_per_worker,), jnp.int32),
            pltpu.VMEM((rows_per_worker, D), jnp.float32),
            pltpu.SemaphoreType.DMA,
        ],
    )
    def gather_k(table_hbm, idx_hbm, out_hbm, idx_v, rows_v, sem):
        wid = lax.axis_index("s") * 2 + lax.axis_index("c")

        @pl.when(wid < n_workers)
        def _():
            base = wid * rows_per_worker
            pltpu.sync_copy(idx_hbm.at[pl.ds(base, rows_per_worker)], idx_v)
            pltpu.async_copy(table_hbm.at[idx_v], rows_v, sem).wait()
            pltpu.sync_copy(rows_v, out_hbm.at[pl.ds(base, rows_per_worker)])

    return gather_k(table, idx_flat)


def kernel(x, W_lin, b_lin, W_lin1, b_lin1, W_max, b_max, W_aug, b_aug,
           gn1_w, gn1_b, gn2_w, gn2_b):
    B, L, D = x.shape
    NT = L // TL
    groups = 32

    wlT = W_lin.T.astype(jnp.bfloat16)
    wl1T = W_lin1.T.astype(jnp.bfloat16)
    wmT = W_max.T.astype(jnp.bfloat16)
    waT = W_aug.T.astype(jnp.bfloat16)
    row = lambda a: a.reshape(1, D)

    x2, colsum, colmax = pl.pallas_call(
        _k1_body,
        grid=(B, NT),
        in_specs=[
            pl.BlockSpec((1, TL, D), lambda b, t: (b, t, 0)),
            pl.BlockSpec((D, D), lambda b, t: (0, 0)),
            pl.BlockSpec((1, D), lambda b, t: (0, 0)),
            pl.BlockSpec((D, D), lambda b, t: (0, 0)),
            pl.BlockSpec((1, D), lambda b, t: (0, 0)),
        ],
        out_specs=[
            pl.BlockSpec((1, TL, D), lambda b, t: (b, t, 0)),
            pl.BlockSpec((1, 1, D), lambda b, t: (b, 0, 0)),
            pl.BlockSpec((1, 1, D), lambda b, t: (b, 0, 0)),
        ],
        out_shape=[
            jax.ShapeDtypeStruct((B, L, D), jnp.bfloat16),
            jax.ShapeDtypeStruct((B, 1, D), jnp.float32),
            jax.ShapeDtypeStruct((B, 1, D), jnp.float32),
        ],
        compiler_params=pltpu.CompilerParams(
            dimension_semantics=("arbitrary", "arbitrary")),
    )(x, wlT, row(b_lin), wl1T, row(b_lin1))

    agg = (jnp.arange(D, dtype=jnp.int32)[:, None] // (D // groups)
           == jnp.arange(groups, dtype=jnp.int32)[None, :]).astype(jnp.float32)
    v = pl.pallas_call(
        functools.partial(_k2_body, L=L, groups=groups, eps=1e-5),
        out_shape=jax.ShapeDtypeStruct((B, 1, D), jnp.float32),
    )(colsum, colmax, wmT, row(b_max), waT, row(b_aug),
      row(gn1_w), row(gn1_b), row(gn2_w), row(gn2_b), agg, agg.T)

    s3 = pl.pallas_call(
        _k3_body,
        grid=(B, NT),
        in_specs=[
            pl.BlockSpec((1, TL, D), lambda b, t: (b, t, 0)),
            pl.BlockSpec((1, 1, D), lambda b, t: (b, 0, 0)),
        ],
        out_specs=pl.BlockSpec((1, TL, 1), lambda b, t: (b, t, 0)),
        out_shape=jax.ShapeDtypeStruct((B, L, 1), jnp.float32),
        compiler_params=pltpu.CompilerParams(
            dimension_semantics=("arbitrary", "arbitrary")),
    )(x2, v)
    s = s3.reshape(B, L)

    flat_idx = pl.pallas_call(
        functools.partial(_k4_body, L=L, k=K),
        out_shape=jax.ShapeDtypeStruct((B, K), jnp.int32),
    )(s)

    x_rows = _sc_gather(x.reshape(B * L, D), flat_idx.reshape(B * K),
                        n_rows=B * K, rows_per_worker=8)
    selected = pl.pallas_call(
        _k6_body,
        out_shape=jax.ShapeDtypeStruct((B * K, D), jnp.float32),
    )(x_rows, wlT, row(b_lin), wl1T, row(b_lin1))
    return selected.reshape(B, K, D)


# speedup vs baseline: 1.1308x; 1.0148x over previous
import functools

import jax
import jax.numpy as jnp
from jax import lax
from jax.experimental import pallas as pl
from jax.experimental.pallas import tpu as pltpu
from jax.experimental.pallas import tpu_sc as plsc

K = 20
TL = 2048


def _k1_body(x_ref, wlT_ref, bl_ref, wl1T_ref, bl1_ref,
             x2_ref, sum_ref, max_ref):
    t = pl.program_id(1)
    x1 = jnp.dot(x_ref[0].astype(jnp.bfloat16), wlT_ref[...],
                 preferred_element_type=jnp.float32) + bl_ref[...]
    part_sum = jnp.sum(x1, axis=0, keepdims=True)
    part_max = jnp.max(x1, axis=0, keepdims=True)

    @pl.when(t == 0)
    def _():
        sum_ref[0] = part_sum
        max_ref[0] = part_max

    @pl.when(t != 0)
    def _():
        sum_ref[0] = sum_ref[0] + part_sum
        max_ref[0] = jnp.maximum(max_ref[0], part_max)

    x2 = jnp.dot(x1.astype(jnp.bfloat16), wl1T_ref[...],
                 preferred_element_type=jnp.float32) + bl1_ref[...]
    x2_ref[0] = x2.astype(jnp.bfloat16)


def _k234_body(x2_ref, sum_ref, max_ref, wmT_ref, bm_ref, waT_ref, ba_ref,
               g1w_ref, g1b_ref, g2w_ref, g2b_ref, agg_ref, aggT_ref,
               idx_ref, q_s, s_s, *, L, NT, groups, eps, k):
    b = pl.program_id(0)
    t = pl.program_id(1)
    TLb = x2_ref.shape[1]
    cs = 1.0 / (agg_ref.shape[0] // groups)
    hi = lax.Precision.HIGHEST

    @pl.when(t == 0)
    def _():
        def gn(y, w, b_):
            gs = jnp.dot(y, agg_ref[...], precision=hi,
                         preferred_element_type=jnp.float32) * cs
            gss = jnp.dot(y * y, agg_ref[...], precision=hi,
                          preferred_element_type=jnp.float32) * cs
            rstd = lax.rsqrt(gss - gs * gs + eps)
            mean_f = jnp.dot(gs, aggT_ref[...], precision=hi,
                             preferred_element_type=jnp.float32)
            rstd_f = jnp.dot(rstd, aggT_ref[...], precision=hi,
                             preferred_element_type=jnp.float32)
            return (y - mean_f) * rstd_f * w + b_

        mean = (sum_ref[0] * (1.0 / L)).astype(jnp.bfloat16)
        xm = jnp.dot(mean, wmT_ref[...],
                     preferred_element_type=jnp.float32) + bm_ref[...]
        xa = jnp.dot(max_ref[0].astype(jnp.bfloat16), waT_ref[...],
                     preferred_element_type=jnp.float32) + ba_ref[...]
        q_s[0:1, :] = gn(xm, g1w_ref[...], g1b_ref[...]).astype(jnp.bfloat16)
        q_s[1:2, :] = gn(xa, g2w_ref[...], g2b_ref[...]).astype(jnp.bfloat16)

    s12 = lax.dot_general(q_s[...], x2_ref[0], (((1,), (1,)), ((), ())),
                          preferred_element_type=jnp.float32)
    s_s[0:1, pl.ds(t * TLb, TLb)] = s12[0:1, :] + s12[1:2, :]

    @pl.when(t == NT - 1)
    def _():
        s = s_s[...]
        col = lax.broadcasted_iota(jnp.int32, (1, L), 1)
        cols = []
        for _ in range(k):
            m = jnp.max(s, axis=1, keepdims=True)
            eq = s == m
            idx = jnp.min(jnp.where(eq, col, L), axis=1, keepdims=True)
            cols.append(idx)
            s = jnp.where(col == idx, -jnp.inf, s)
        idx_ref[0] = jnp.concatenate(cols, axis=1) + b * L


def _k6_body(xr_ref, wlT_ref, bl_ref, wl1T_ref, bl1_ref, out_ref):
    x1 = jnp.dot(xr_ref[...].astype(jnp.bfloat16), wlT_ref[...],
                 preferred_element_type=jnp.float32) + bl_ref[...]
    out_ref[...] = jnp.dot(x1.astype(jnp.bfloat16), wl1T_ref[...],
                           preferred_element_type=jnp.float32) + bl1_ref[...]


def _sc_gather(table, idx_flat, n_rows, rows_per_worker):
    D = table.shape[1]
    n_workers = n_rows // rows_per_worker
    mesh = plsc.VectorSubcoreMesh(core_axis_name="c", subcore_axis_name="s")

    @functools.partial(
        pl.kernel, mesh=mesh,
        out_type=jax.ShapeDtypeStruct((n_rows, D), jnp.float32),
        scratch_types=[
            pltpu.VMEM((rows_per_worker,), jnp.int32),
            pltpu.VMEM((rows_per_worker, D), jnp.float32),
            pltpu.SemaphoreType.DMA,
        ],
    )
    def gather_k(table_hbm, idx_hbm, out_hbm, idx_v, rows_v, sem):
        wid = lax.axis_index("s") * 2 + lax.axis_index("c")

        @pl.when(wid < n_workers)
        def _():
            base = wid * rows_per_worker
            pltpu.sync_copy(idx_hbm.at[pl.ds(base, rows_per_worker)], idx_v)
            pltpu.async_copy(table_hbm.at[idx_v], rows_v, sem).wait()
            pltpu.sync_copy(rows_v, out_hbm.at[pl.ds(base, rows_per_worker)])

    return gather_k(table, idx_flat)


def kernel(x, W_lin, b_lin, W_lin1, b_lin1, W_max, b_max, W_aug, b_aug,
           gn1_w, gn1_b, gn2_w, gn2_b):
    B, L, D = x.shape
    NT = L // TL
    groups = 32

    wlT = W_lin.T.astype(jnp.bfloat16)
    wl1T = W_lin1.T.astype(jnp.bfloat16)
    wmT = W_max.T.astype(jnp.bfloat16)
    waT = W_aug.T.astype(jnp.bfloat16)
    row = lambda a: a.reshape(1, D)

    x2, colsum, colmax = pl.pallas_call(
        _k1_body,
        grid=(B, NT),
        in_specs=[
            pl.BlockSpec((1, TL, D), lambda b, t: (b, t, 0)),
            pl.BlockSpec((D, D), lambda b, t: (0, 0)),
            pl.BlockSpec((1, D), lambda b, t: (0, 0)),
            pl.BlockSpec((D, D), lambda b, t: (0, 0)),
            pl.BlockSpec((1, D), lambda b, t: (0, 0)),
        ],
        out_specs=[
            pl.BlockSpec((1, TL, D), lambda b, t: (b, t, 0)),
            pl.BlockSpec((1, 1, D), lambda b, t: (b, 0, 0)),
            pl.BlockSpec((1, 1, D), lambda b, t: (b, 0, 0)),
        ],
        out_shape=[
            jax.ShapeDtypeStruct((B, L, D), jnp.bfloat16),
            jax.ShapeDtypeStruct((B, 1, D), jnp.float32),
            jax.ShapeDtypeStruct((B, 1, D), jnp.float32),
        ],
        compiler_params=pltpu.CompilerParams(
            dimension_semantics=("arbitrary", "arbitrary")),
    )(x, wlT, row(b_lin), wl1T, row(b_lin1))

    agg = (jnp.arange(D, dtype=jnp.int32)[:, None] // (D // groups)
           == jnp.arange(groups, dtype=jnp.int32)[None, :]).astype(jnp.float32)
    flat_idx = pl.pallas_call(
        functools.partial(_k234_body, L=L, NT=NT, groups=groups, eps=1e-5,
                          k=K),
        grid=(B, NT),
        in_specs=[
            pl.BlockSpec((1, TL, D), lambda b, t: (b, t, 0)),
            pl.BlockSpec((1, 1, D), lambda b, t: (b, 0, 0)),
            pl.BlockSpec((1, 1, D), lambda b, t: (b, 0, 0)),
            pl.BlockSpec((D, D), lambda b, t: (0, 0)),
            pl.BlockSpec((1, D), lambda b, t: (0, 0)),
            pl.BlockSpec((D, D), lambda b, t: (0, 0)),
            pl.BlockSpec((1, D), lambda b, t: (0, 0)),
            pl.BlockSpec((1, D), lambda b, t: (0, 0)),
            pl.BlockSpec((1, D), lambda b, t: (0, 0)),
            pl.BlockSpec((1, D), lambda b, t: (0, 0)),
            pl.BlockSpec((1, D), lambda b, t: (0, 0)),
            pl.BlockSpec((D, groups), lambda b, t: (0, 0)),
            pl.BlockSpec((groups, D), lambda b, t: (0, 0)),
        ],
        out_specs=pl.BlockSpec((1, 1, K), lambda b, t: (b, 0, 0)),
        out_shape=jax.ShapeDtypeStruct((B, 1, K), jnp.int32),
        scratch_shapes=[
            pltpu.VMEM((8, D), jnp.bfloat16),
            pltpu.VMEM((1, L), jnp.float32),
        ],
        compiler_params=pltpu.CompilerParams(
            dimension_semantics=("arbitrary", "arbitrary")),
    )(x2, colsum, colmax, wmT, row(b_max), waT, row(b_aug),
      row(gn1_w), row(gn1_b), row(gn2_w), row(gn2_b), agg, agg.T)

    x_rows = _sc_gather(x.reshape(B * L, D), flat_idx.reshape(B * K),
                        n_rows=B * K, rows_per_worker=8)
    selected = pl.pallas_call(
        _k6_body,
        out_shape=jax.ShapeDtypeStruct((B * K, D), jnp.float32),
    )(x_rows, wlT, row(b_lin), wl1T, row(b_lin1))
    return selected.reshape(B, K, D)
